# baseline (speedup 1.0000x reference)
import functools

import jax
import jax.numpy as jnp
from jax import lax
from jax.experimental import pallas as pl
from jax.experimental.pallas import tpu as pltpu
from jax.experimental.pallas import tpu_sc as plsc

_E, _K, _NS = 8, 2, 1
_AUX = 0.01
_T, _H, _I = 2048, 1024, 1024
_BLK = 256
_PR = ((_T * _K + _E * (_BLK - 1)) + _BLK - 1) // _BLK * _BLK
_NBR = _PR // _BLK
_NW = 32
_CHUNK = _T // _NW


def _router_body(x_ref, gw_ref, w0_ref, w1_ref, pos0_ref, pos1_ref,
                 be_ref, bv_ref, bx_ref, loss_ref):
    x = x_ref[...]
    logits = lax.dot_general(gw_ref[...], x, (((0,), (1,)), ((), ())),
                             preferred_element_type=jnp.float32)
    m = jnp.max(logits, axis=0, keepdims=True)
    ex = jnp.exp(logits - m)
    p = ex / jnp.sum(ex, axis=0, keepdims=True)
    t = x.shape[0]
    row = lax.broadcasted_iota(jnp.int32, (_E, t), 0)
    rank = jnp.zeros((_E, t), jnp.float32)
    for j in range(_E):
        pj = p[j:j + 1, :]
        rank += (pj > p).astype(jnp.float32)
        rank += ((pj == p) & (row > j)).astype(jnp.float32)
    sel = rank < _K
    sel_f = sel.astype(jnp.float32)
    selp = jnp.where(sel, p, 0.0)
    w = selp / jnp.sum(selp, axis=0, keepdims=True)
    tpe = jnp.sum(sel_f, axis=1, keepdims=True)
    ppe = jnp.mean(p, axis=1, keepdims=True)
    loss_ref[...] = jnp.sum(tpe * ppe, keepdims=True) * (_E * _AUX / t)
    cume = sel_f
    sh = 1
    while sh < t:
        left = jnp.zeros((_E, sh), jnp.float32)
        cume = cume + jnp.concatenate([left, cume[:, :t - sh]], axis=1)
        sh *= 2
    cume = cume - sel_f
    run = sel_f
    sh = 1
    while sh < _E:
        top = jnp.zeros((sh, t), jnp.float32)
        run = run + jnp.concatenate([top, run[:_E - sh, :]], axis=0)
        sh *= 2
    csel = run
    first = sel & (csel == 1.0)
    second = sel & (csel == 2.0)
    cnt = tpe
    padded = ((cnt.astype(jnp.int32) + (_BLK - 1)) // _BLK) * _BLK
    padf = padded.astype(jnp.float32)
    offr = padf
    sh = 1
    while sh < _E:
        top = jnp.zeros((sh, 1), jnp.float32)
        offr = offr + jnp.concatenate([top, offr[:_E - sh, :]], axis=0)
        sh *= 2
    off = offr - padf
    slot = off + cume
    w0_ref[...] = jnp.sum(jnp.where(first, w, 0.0), axis=0, keepdims=True)
    w1_ref[...] = jnp.sum(jnp.where(second, w, 0.0), axis=0, keepdims=True)
    pos0_ref[...] = jnp.sum(jnp.where(first, slot, 0.0), axis=0,
                            keepdims=True).astype(jnp.int32)
    pos1_ref[...] = jnp.sum(jnp.where(second, slot, 0.0), axis=0,
                            keepdims=True).astype(jnp.int32)
    total = jnp.sum(padded)
    sb = lax.broadcasted_iota(jnp.int32, (1, _NBR), 1) * _BLK
    off_i = off.astype(jnp.int32)
    acc = jnp.sum((sb >= off_i).astype(jnp.int32), axis=0, keepdims=True)
    be_ref[...] = jnp.maximum(acc - 1, 0)
    bv_ref[...] = (sb < total).astype(jnp.int32)
    bx_ref[...] = jnp.minimum(sb // _BLK, total // _BLK - 1)


def _router(x, gate_w):
    return pl.pallas_call(
        _router_body,
        out_shape=[
            jax.ShapeDtypeStruct((1, _T), jnp.float32),
            jax.ShapeDtypeStruct((1, _T), jnp.float32),
            jax.ShapeDtypeStruct((1, _T), jnp.int32),
            jax.ShapeDtypeStruct((1, _T), jnp.int32),
            jax.ShapeDtypeStruct((1, _NBR), jnp.int32),
            jax.ShapeDtypeStruct((1, _NBR), jnp.int32),
            jax.ShapeDtypeStruct((1, _NBR), jnp.int32),
            jax.ShapeDtypeStruct((1, 1), jnp.float32),
        ],
    )(x, gate_w)


@functools.cache
def _make_dispatch_sc():
    mesh = plsc.VectorSubcoreMesh(core_axis_name="c", subcore_axis_name="s")

    @functools.partial(
        pl.kernel,
        out_type=jax.ShapeDtypeStruct((_PR, _H), jnp.float32),
        mesh=mesh,
        scratch_types=[
            pltpu.VMEM((_CHUNK,), jnp.int32),
            pltpu.VMEM((_CHUNK,), jnp.int32),
            pltpu.VMEM((_CHUNK, _H), jnp.float32),
            pltpu.SemaphoreType.DMA,
        ],
    )
    def dispatch_sc(x_hbm, pos0_hbm, pos1_hbm, xs_hbm, idx0_v, idx1_v, rows_v, sem):
        wid = lax.axis_index("s") * 2 + lax.axis_index("c")
        base = wid * _CHUNK
        pltpu.sync_copy(x_hbm.at[pl.ds(base, _CHUNK)], rows_v)
        pltpu.sync_copy(pos0_hbm.at[pl.ds(base, _CHUNK)], idx0_v)
        pltpu.sync_copy(pos1_hbm.at[pl.ds(base, _CHUNK)], idx1_v)
        pltpu.async_copy(rows_v, xs_hbm.at[idx0_v], sem).wait()
        pltpu.async_copy(rows_v, xs_hbm.at[idx1_v], sem).wait()

    return dispatch_sc


def _dispatch(x, pos0, pos1):
    return _make_dispatch_sc()(x, pos0, pos1)


@functools.cache
def _make_combine_sc():
    mesh = plsc.VectorSubcoreMesh(core_axis_name="c", subcore_axis_name="s")

    @functools.partial(
        pl.kernel,
        out_type=(jax.ShapeDtypeStruct((_T, _H), jnp.float32),
                  jax.ShapeDtypeStruct((_T, _H), jnp.float32)),
        mesh=mesh,
        scratch_types=[
            pltpu.VMEM((_CHUNK,), jnp.int32),
            pltpu.VMEM((_CHUNK, _H), jnp.float32),
            pltpu.SemaphoreType.DMA,
        ],
    )
    def combine_sc(ys_hbm, pos0_hbm, pos1_hbm, y0_hbm, y1_hbm, idx_v, rows_v, sem):
        wid = lax.axis_index("s") * 2 + lax.axis_index("c")
        base = wid * _CHUNK
        pltpu.sync_copy(pos0_hbm.at[pl.ds(base, _CHUNK)], idx_v)
        pltpu.async_copy(ys_hbm.at[idx_v], rows_v, sem).wait()
        pltpu.sync_copy(rows_v, y0_hbm.at[pl.ds(base, _CHUNK)])
        pltpu.sync_copy(pos1_hbm.at[pl.ds(base, _CHUNK)], idx_v)
        pltpu.async_copy(ys_hbm.at[idx_v], rows_v, sem).wait()
        pltpu.sync_copy(rows_v, y1_hbm.at[pl.ds(base, _CHUNK)])

    return combine_sc


def _combine(ys, pos0, pos1):
    return _make_combine_sc()(ys, pos0, pos1)


def _gffn_body(be_sref, bv_sref, bx_sref, xs_ref, wg_ref, wu_ref, wd_ref,
               ys_ref):
    b = pl.program_id(0)

    @pl.when(bv_sref[b] != 0)
    def _():
        xb = xs_ref[...]
        g = jnp.dot(xb, wg_ref[0], preferred_element_type=jnp.float32)
        u = jnp.dot(xb, wu_ref[0], preferred_element_type=jnp.float32)
        h = g * jax.nn.sigmoid(g) * u
        ys_ref[...] = jnp.dot(h, wd_ref[0], preferred_element_type=jnp.float32)


def _grouped_ffn(be, bv, bx, xs, wg, wu, wd):
    grid_spec = pltpu.PrefetchScalarGridSpec(
        num_scalar_prefetch=3,
        grid=(_NBR,),
        in_specs=[
            pl.BlockSpec((_BLK, _H), lambda b, be, bv, bx: (bx[b], 0)),
            pl.BlockSpec((1, _H, _I), lambda b, be, bv, bx: (be[b], 0, 0)),
            pl.BlockSpec((1, _H, _I), lambda b, be, bv, bx: (be[b], 0, 0)),
            pl.BlockSpec((1, _I, _H), lambda b, be, bv, bx: (be[b], 0, 0)),
        ],
        out_specs=pl.BlockSpec((_BLK, _H), lambda b, be, bv, bx: (bx[b], 0)),
    )
    return pl.pallas_call(
        _gffn_body,
        grid_spec=grid_spec,
        out_shape=jax.ShapeDtypeStruct((_PR, _H), jnp.float32),
    )(be, bv, bx, xs, wg, wu, wd)


def _shared_body(x_ref, sg_ref, su_ref, sd_ref, out_ref):
    x = x_ref[...]
    g = jnp.dot(x, sg_ref[0], preferred_element_type=jnp.float32)
    u = jnp.dot(x, su_ref[0], preferred_element_type=jnp.float32)
    h = g * jax.nn.sigmoid(g) * u
    ysh = jnp.dot(h, sd_ref[0], preferred_element_type=jnp.float32)
    out_ref[...] = ysh * (1.0 / _NS)


def _shared_ffn_half(x, sg, su, sd, half):
    tb = 256
    ih = _I // 2
    return pl.pallas_call(
        _shared_body,
        grid=(_T // tb,),
        in_specs=[
            pl.BlockSpec((tb, _H), lambda t: (t, 0)),
            pl.BlockSpec((1, _H, ih), lambda t: (0, 0, half)),
            pl.BlockSpec((1, _H, ih), lambda t: (0, 0, half)),
            pl.BlockSpec((1, ih, _H), lambda t: (0, half, 0)),
        ],
        out_specs=pl.BlockSpec((tb, _H), lambda t: (t, 0)),
        out_shape=jax.ShapeDtypeStruct((_T, _H), jnp.float32),
    )(x, sg, su, sd)


def _final_body(sh0_ref, sh1_ref, y0_ref, y1_ref, w0_ref, w1_ref, out_ref):
    out_ref[...] = (sh0_ref[...] + sh1_ref[...]
                    + w0_ref[...] * y0_ref[...] + w1_ref[...] * y1_ref[...])


def _final_combine(sh0, sh1, y0, y1, w0, w1):
    tb = 512
    return pl.pallas_call(
        _final_body,
        grid=(_T // tb,),
        in_specs=[
            pl.BlockSpec((tb, _H), lambda t: (t, 0)),
            pl.BlockSpec((tb, _H), lambda t: (t, 0)),
            pl.BlockSpec((tb, _H), lambda t: (t, 0)),
            pl.BlockSpec((tb, _H), lambda t: (t, 0)),
            pl.BlockSpec((tb, 1), lambda t: (t, 0)),
            pl.BlockSpec((tb, 1), lambda t: (t, 0)),
        ],
        out_specs=pl.BlockSpec((tb, _H), lambda t: (t, 0)),
        out_shape=jax.ShapeDtypeStruct((_T, _H), jnp.float32),
    )(sh0, sh1, y0, y1, w0, w1)


def kernel(hidden_states, gate_W, Wg, Wu, Wd, Sg, Su, Sd):
    b, s, h = hidden_states.shape
    x = hidden_states.reshape(s, h)
    w0, w1, pos0, pos1, be, bv, bx, loss = _router(x, gate_W)
    pos0f = pos0.reshape(s)
    pos1f = pos1.reshape(s)
    xs = _dispatch(x, pos0f, pos1f)
    sh0 = _shared_ffn_half(x, Sg, Su, Sd, 0)
    ys = _grouped_ffn(be.reshape(_NBR), bv.reshape(_NBR), bx.reshape(_NBR),
                      xs, Wg, Wu, Wd)
    y0, y1 = _combine(ys, pos0f, pos1f)
    sh1 = _shared_ffn_half(x, Sg, Su, Sd, 1)
    out = _final_combine(sh0, sh1, y0, y1,
                         w0.reshape(s, 1), w1.reshape(s, 1))
    return out.reshape(b, s, h), loss[0, 0]


# speedup vs baseline: 1.0577x; 1.0577x over previous
import functools

import jax
import jax.numpy as jnp
from jax import lax
from jax.experimental import pallas as pl
from jax.experimental.pallas import tpu as pltpu
from jax.experimental.pallas import tpu_sc as plsc

_E, _K, _NS = 8, 2, 1
_AUX = 0.01
_T, _H, _I = 2048, 1024, 1024
_BLK = 256
_PR = ((_T * _K + _E * (_BLK - 1)) + _BLK - 1) // _BLK * _BLK
_NBR = _PR // _BLK
_NW = 32
_CHUNK = _T // _NW


def _router_body(x_ref, gw_ref, w0_ref, w1_ref, pos0_ref, pos1_ref,
                 be_ref, bv_ref, bx_ref, loss_ref):
    x = x_ref[...]
    logits = lax.dot_general(gw_ref[...], x, (((0,), (1,)), ((), ())),
                             preferred_element_type=jnp.float32)
    m = jnp.max(logits, axis=0, keepdims=True)
    ex = jnp.exp(logits - m)
    p = ex / jnp.sum(ex, axis=0, keepdims=True)
    t = x.shape[0]
    row = lax.broadcasted_iota(jnp.int32, (_E, t), 0)
    rank = jnp.zeros((_E, t), jnp.float32)
    for j in range(_E):
        pj = p[j:j + 1, :]
        rank += (pj > p).astype(jnp.float32)
        rank += ((pj == p) & (row > j)).astype(jnp.float32)
    sel = rank < _K
    sel_f = sel.astype(jnp.float32)
    selp = jnp.where(sel, p, 0.0)
    w = selp / jnp.sum(selp, axis=0, keepdims=True)
    tpe = jnp.sum(sel_f, axis=1, keepdims=True)
    ppe = jnp.mean(p, axis=1, keepdims=True)
    loss_ref[...] = jnp.sum(tpe * ppe, keepdims=True) * (_E * _AUX / t)
    cume = sel_f
    sh = 1
    while sh < t:
        left = jnp.zeros((_E, sh), jnp.float32)
        cume = cume + jnp.concatenate([left, cume[:, :t - sh]], axis=1)
        sh *= 2
    cume = cume - sel_f
    run = sel_f
    sh = 1
    while sh < _E:
        top = jnp.zeros((sh, t), jnp.float32)
        run = run + jnp.concatenate([top, run[:_E - sh, :]], axis=0)
        sh *= 2
    csel = run
    first = sel & (csel == 1.0)
    second = sel & (csel == 2.0)
    cnt = tpe
    padded = ((cnt.astype(jnp.int32) + (_BLK - 1)) // _BLK) * _BLK
    padf = padded.astype(jnp.float32)
    offr = padf
    sh = 1
    while sh < _E:
        top = jnp.zeros((sh, 1), jnp.float32)
        offr = offr + jnp.concatenate([top, offr[:_E - sh, :]], axis=0)
        sh *= 2
    off = offr - padf
    slot = off + cume
    w0_ref[...] = jnp.sum(jnp.where(first, w, 0.0), axis=0, keepdims=True)
    w1_ref[...] = jnp.sum(jnp.where(second, w, 0.0), axis=0, keepdims=True)
    pos0_ref[...] = jnp.sum(jnp.where(first, slot, 0.0), axis=0,
                            keepdims=True).astype(jnp.int32)
    pos1_ref[...] = jnp.sum(jnp.where(second, slot, 0.0), axis=0,
                            keepdims=True).astype(jnp.int32)
    total = jnp.sum(padded)
    sb = lax.broadcasted_iota(jnp.int32, (1, _NBR), 1) * _BLK
    off_i = off.astype(jnp.int32)
    acc = jnp.sum((sb >= off_i).astype(jnp.int32), axis=0, keepdims=True)
    be_ref[...] = jnp.maximum(acc - 1, 0)
    bv_ref[...] = (sb < total).astype(jnp.int32)
    bx_ref[...] = jnp.minimum(sb // _BLK, total // _BLK - 1)


def _router(x, gate_w):
    return pl.pallas_call(
        _router_body,
        out_shape=[
            jax.ShapeDtypeStruct((1, _T), jnp.float32),
            jax.ShapeDtypeStruct((1, _T), jnp.float32),
            jax.ShapeDtypeStruct((1, _T), jnp.int32),
            jax.ShapeDtypeStruct((1, _T), jnp.int32),
            jax.ShapeDtypeStruct((1, _NBR), jnp.int32),
            jax.ShapeDtypeStruct((1, _NBR), jnp.int32),
            jax.ShapeDtypeStruct((1, _NBR), jnp.int32),
            jax.ShapeDtypeStruct((1, 1), jnp.float32),
        ],
    )(x, gate_w)


@functools.cache
def _make_dispatch_sc():
    mesh = plsc.VectorSubcoreMesh(core_axis_name="c", subcore_axis_name="s")

    @functools.partial(
        pl.kernel,
        out_type=jax.ShapeDtypeStruct((_PR, _H), jnp.float32),
        mesh=mesh,
        scratch_types=[
            pltpu.VMEM((_CHUNK,), jnp.int32),
            pltpu.VMEM((_CHUNK,), jnp.int32),
            pltpu.VMEM((_CHUNK, _H), jnp.float32),
            pltpu.SemaphoreType.DMA,
        ],
    )
    def dispatch_sc(x_hbm, pos0_hbm, pos1_hbm, xs_hbm, idx0_v, idx1_v, rows_v, sem):
        wid = lax.axis_index("s") * 2 + lax.axis_index("c")
        base = wid * _CHUNK
        pltpu.sync_copy(x_hbm.at[pl.ds(base, _CHUNK)], rows_v)
        pltpu.sync_copy(pos0_hbm.at[pl.ds(base, _CHUNK)], idx0_v)
        pltpu.sync_copy(pos1_hbm.at[pl.ds(base, _CHUNK)], idx1_v)
        pltpu.async_copy(rows_v, xs_hbm.at[idx0_v], sem).wait()
        pltpu.async_copy(rows_v, xs_hbm.at[idx1_v], sem).wait()

    return dispatch_sc


def _dispatch(x, pos0, pos1):
    return _make_dispatch_sc()(x, pos0, pos1)


@functools.cache
def _make_combine_sc():
    mesh = plsc.VectorSubcoreMesh(core_axis_name="c", subcore_axis_name="s")

    @functools.partial(
        pl.kernel,
        out_type=(jax.ShapeDtypeStruct((_T, _H), jnp.float32),
                  jax.ShapeDtypeStruct((_T, _H), jnp.float32)),
        mesh=mesh,
        scratch_types=[
            pltpu.VMEM((_CHUNK,), jnp.int32),
            pltpu.VMEM((_CHUNK, _H), jnp.float32),
            pltpu.SemaphoreType.DMA,
        ],
    )
    def combine_sc(ys_hbm, pos0_hbm, pos1_hbm, y0_hbm, y1_hbm, idx_v, rows_v, sem):
        wid = lax.axis_index("s") * 2 + lax.axis_index("c")
        base = wid * _CHUNK
        pltpu.sync_copy(pos0_hbm.at[pl.ds(base, _CHUNK)], idx_v)
        pltpu.async_copy(ys_hbm.at[idx_v], rows_v, sem).wait()
        pltpu.sync_copy(rows_v, y0_hbm.at[pl.ds(base, _CHUNK)])
        pltpu.sync_copy(pos1_hbm.at[pl.ds(base, _CHUNK)], idx_v)
        pltpu.async_copy(ys_hbm.at[idx_v], rows_v, sem).wait()
        pltpu.sync_copy(rows_v, y1_hbm.at[pl.ds(base, _CHUNK)])

    return combine_sc


def _combine(ys, pos0, pos1):
    return _make_combine_sc()(ys, pos0, pos1)


def _gffn_body(be_sref, bv_sref, bx_sref, xs_ref, wg_ref, wu_ref, wd_ref,
               ys_ref):
    b = pl.program_id(0)

    @pl.when(bv_sref[b] != 0)
    def _():
        xb = xs_ref[...]
        g = jnp.dot(xb, wg_ref[0], preferred_element_type=jnp.float32)
        u = jnp.dot(xb, wu_ref[0], preferred_element_type=jnp.float32)
        h = g * jax.nn.sigmoid(g) * u
        ys_ref[...] = jnp.dot(h, wd_ref[0], preferred_element_type=jnp.float32)


def _grouped_ffn(be, bv, bx, xs, wg, wu, wd):
    grid_spec = pltpu.PrefetchScalarGridSpec(
        num_scalar_prefetch=3,
        grid=(_NBR,),
        in_specs=[
            pl.BlockSpec((_BLK, _H), lambda b, be, bv, bx: (bx[b], 0)),
            pl.BlockSpec((1, _H, _I), lambda b, be, bv, bx: (be[b], 0, 0)),
            pl.BlockSpec((1, _H, _I), lambda b, be, bv, bx: (be[b], 0, 0)),
            pl.BlockSpec((1, _I, _H), lambda b, be, bv, bx: (be[b], 0, 0)),
        ],
        out_specs=pl.BlockSpec((_BLK, _H), lambda b, be, bv, bx: (bx[b], 0)),
    )
    return pl.pallas_call(
        _gffn_body,
        grid_spec=grid_spec,
        out_shape=jax.ShapeDtypeStruct((_PR, _H), jnp.float32),
    )(be, bv, bx, xs, wg, wu, wd)


def _shared_body(x_ref, sg_ref, su_ref, sd_ref, out_ref):
    x = x_ref[...]
    g = jnp.dot(x, sg_ref[0], preferred_element_type=jnp.float32)
    u = jnp.dot(x, su_ref[0], preferred_element_type=jnp.float32)
    h = g * jax.nn.sigmoid(g) * u
    ysh = jnp.dot(h, sd_ref[0], preferred_element_type=jnp.float32)
    out_ref[...] = ysh * (1.0 / _NS)


def _shared_ffn(x, sg, su, sd):
    tb = 256
    return pl.pallas_call(
        _shared_body,
        grid=(_T // tb,),
        in_specs=[
            pl.BlockSpec((tb, _H), lambda t: (t, 0)),
            pl.BlockSpec((1, _H, _I), lambda t: (0, 0, 0)),
            pl.BlockSpec((1, _H, _I), lambda t: (0, 0, 0)),
            pl.BlockSpec((1, _I, _H), lambda t: (0, 0, 0)),
        ],
        out_specs=pl.BlockSpec((tb, _H), lambda t: (t, 0)),
        out_shape=jax.ShapeDtypeStruct((_T, _H), jnp.float32),
    )(x, sg, su, sd)


def _final_body(sh_ref, y0_ref, y1_ref, w0_ref, w1_ref, out_ref):
    out_ref[...] = (sh_ref[...]
                    + w0_ref[...] * y0_ref[...] + w1_ref[...] * y1_ref[...])


def _final_combine(sh, y0, y1, w0, w1):
    tb = 512
    return pl.pallas_call(
        _final_body,
        grid=(_T // tb,),
        in_specs=[
            pl.BlockSpec((tb, _H), lambda t: (t, 0)),
            pl.BlockSpec((tb, _H), lambda t: (t, 0)),
            pl.BlockSpec((tb, _H), lambda t: (t, 0)),
            pl.BlockSpec((tb, 1), lambda t: (t, 0)),
            pl.BlockSpec((tb, 1), lambda t: (t, 0)),
        ],
        out_specs=pl.BlockSpec((tb, _H), lambda t: (t, 0)),
        out_shape=jax.ShapeDtypeStruct((_T, _H), jnp.float32),
    )(sh, y0, y1, w0, w1)


def kernel(hidden_states, gate_W, Wg, Wu, Wd, Sg, Su, Sd):
    b, s, h = hidden_states.shape
    x = hidden_states.reshape(s, h)
    w0, w1, pos0, pos1, be, bv, bx, loss = _router(x, gate_W)
    pos0f = pos0.reshape(s)
    pos1f = pos1.reshape(s)
    xs = _dispatch(x, pos0f, pos1f)
    ys = _grouped_ffn(be.reshape(_NBR), bv.reshape(_NBR), bx.reshape(_NBR),
                      xs, Wg, Wu, Wd)
    y0, y1 = _combine(ys, pos0f, pos1f)
    sh = _shared_ffn(x, Sg, Su, Sd)
    out = _final_combine(sh, y0, y1,
                         w0.reshape(s, 1), w1.reshape(s, 1))
    return out.reshape(b, s, h), loss[0, 0]
